# baseline (speedup 1.0000x reference)
import jax
import jax.numpy as jnp
from jax.experimental import pallas as pl
from jax.experimental.pallas import tpu as pltpu

_ALPHA = 0.1
_BETA = 0.5

_BM = 80
_NBUF = 8


def _make_body(n, k, f, fo, nblocks):
    def body(g_hbm, f_hbm, f0_hbm, w_ref, b_ref, o_hbm,
             f_vmem, f0_vmem, g_ring, o_ring, g_sem, o_sem, f_sem, f0_sem):
        def g_copy(j, slot):
            return pltpu.make_async_copy(
                g_hbm.at[pl.ds(j * _BM, _BM), :], g_ring.at[slot], g_sem.at[slot])

        def o_copy(j, slot):
            return pltpu.make_async_copy(
                o_ring.at[slot], o_hbm.at[pl.ds(j * _BM, _BM), :], o_sem.at[slot])

        for s in range(_NBUF):
            g_copy(s, s).start()
        pltpu.make_async_copy(f_hbm, f_vmem, f_sem).start()
        pltpu.make_async_copy(f0_hbm, f0_vmem, f0_sem).start()

        def step(j, carry):
            slot = jax.lax.rem(j, _NBUF)
            g_copy(j, slot).wait()

            @pl.when(j == 0)
            def _wait_consts():
                pltpu.make_async_copy(f_hbm, f_vmem, f_sem).wait()
                pltpu.make_async_copy(f0_hbm, f0_vmem, f0_sem).wait()

            hi = jnp.dot(g_ring[slot], f_vmem[...],
                         preferred_element_type=jnp.float32)
            support = (1.0 - _ALPHA) * hi + _ALPHA * f0_vmem[pl.ds(j * _BM, _BM), :]
            out = _BETA * jnp.dot(support, w_ref[...],
                                  preferred_element_type=jnp.float32)
            out = out + (1.0 - _BETA) * support + b_ref[...]

            @pl.when(j >= _NBUF)
            def _recycle_out():
                o_copy(j - _NBUF, slot).wait()

            o_ring[slot] = out
            o_copy(j, slot).start()

            @pl.when(j + _NBUF < nblocks)
            def _refill():
                g_copy(j + _NBUF, slot).start()

            return carry

        jax.lax.fori_loop(0, nblocks, step, 0)
        for s in range(_NBUF):
            j = nblocks - _NBUF + s
            o_copy(j, j % _NBUF).wait()

    return body


def kernel(graph, features, features0, w, b):
    n, k = graph.shape
    f = features.shape[1]
    fo = w.shape[1]
    b2 = b.reshape(1, fo)
    nblocks = n // _BM
    assert n % _BM == 0 and nblocks >= _NBUF

    return pl.pallas_call(
        _make_body(n, k, f, fo, nblocks),
        in_specs=[
            pl.BlockSpec(memory_space=pltpu.MemorySpace.HBM),
            pl.BlockSpec(memory_space=pltpu.MemorySpace.HBM),
            pl.BlockSpec(memory_space=pltpu.MemorySpace.HBM),
            pl.BlockSpec(memory_space=pltpu.MemorySpace.VMEM),
            pl.BlockSpec(memory_space=pltpu.MemorySpace.VMEM),
        ],
        out_specs=pl.BlockSpec(memory_space=pltpu.MemorySpace.HBM),
        out_shape=jax.ShapeDtypeStruct((n, fo), jnp.float32),
        scratch_shapes=[
            pltpu.VMEM((k, f), jnp.float32),
            pltpu.VMEM((n, f), jnp.float32),
            pltpu.VMEM((_NBUF, _BM, k), jnp.float32),
            pltpu.VMEM((_NBUF, _BM, fo), jnp.float32),
            pltpu.SemaphoreType.DMA((_NBUF,)),
            pltpu.SemaphoreType.DMA((_NBUF,)),
            pltpu.SemaphoreType.DMA,
            pltpu.SemaphoreType.DMA,
        ],
    )(graph, features, features0, w, b2)
